# baseline (speedup 1.0000x reference)
import functools

import jax
import jax.numpy as jnp
from jax import lax
from jax.experimental import pallas as pl

T, D, EK, S = 2048, 1024, 128, 32
TB = 512


def _body(x_ref, idxt_ref, u_ref, svh_ref, o_ref):
    p_t = lax.dot_general(svh_ref[...], x_ref[...], (((1,), (1,)), ((), ())),
                          preferred_element_type=jnp.float32)
    jota = lax.broadcasted_iota(jnp.int32, (EK, TB), 0).astype(jnp.int16)
    cnt_t = jnp.zeros((EK, TB), jnp.int16)
    for s in range(S):
        cnt_t = cnt_t + (idxt_ref[s:s + 1, :] == jota).astype(jnp.int16)
    scaled_t = p_t * cnt_t.astype(jnp.float32)
    o_ref[...] = lax.dot_general(scaled_t, u_ref[...], (((0,), (1,)), ((), ())),
                                 preferred_element_type=jnp.float32)


@jax.jit
def _run(x, index, u, svh):
    idxt = index.T.astype(jnp.int16)
    return pl.pallas_call(
        _body,
        grid=(T // TB,),
        in_specs=[
            pl.BlockSpec((TB, D), lambda i: (i, 0)),
            pl.BlockSpec((S, TB), lambda i: (0, i)),
            pl.BlockSpec((D, EK), lambda i: (0, 0)),
            pl.BlockSpec((EK, D), lambda i: (0, 0)),
        ],
        out_specs=pl.BlockSpec((TB, D), lambda i: (i, 0)),
        out_shape=jax.ShapeDtypeStruct((T, D), jnp.float32),
    )(x, idxt, u, svh)


def kernel(x, routing_weights, index, u, svh):
    del routing_weights
    return _run(x, index, u, svh)


# speedup vs baseline: 1.2208x; 1.2208x over previous
import functools

import jax
import jax.numpy as jnp
from jax import lax
from jax.experimental import pallas as pl
from jax.experimental.pallas import tpu as pltpu

T, D, EK, S = 2048, 1024, 128, 32
TB = 1024


def _body(x_ref, idxt_ref, u_ref, svh_ref, o_ref):
    p_t = lax.dot_general(svh_ref[...], x_ref[...], (((1,), (1,)), ((), ())),
                          preferred_element_type=jnp.float32)
    jota = lax.broadcasted_iota(jnp.int32, (EK, TB), 0).astype(jnp.int16)
    cnt_t = jnp.zeros((EK, TB), jnp.int16)
    for s in range(S):
        cnt_t = cnt_t + (idxt_ref[s:s + 1, :] == jota).astype(jnp.int16)
    scaled_t = p_t * cnt_t.astype(jnp.float32)
    o_ref[...] = lax.dot_general(scaled_t, u_ref[...], (((0,), (1,)), ((), ())),
                                 preferred_element_type=jnp.float32)


@jax.jit
def _run(x, index, u, svh):
    idxt = index.T.astype(jnp.int16)
    return pl.pallas_call(
        _body,
        grid=(T // TB,),
        compiler_params=pltpu.CompilerParams(allow_input_fusion=[False, True, False, False]),
        in_specs=[
            pl.BlockSpec((TB, D), lambda i: (i, 0)),
            pl.BlockSpec((S, TB), lambda i: (0, i)),
            pl.BlockSpec((D, EK), lambda i: (0, 0)),
            pl.BlockSpec((EK, D), lambda i: (0, 0)),
        ],
        out_specs=pl.BlockSpec((TB, D), lambda i: (i, 0)),
        out_shape=jax.ShapeDtypeStruct((T, D), jnp.float32),
    )(x, idxt, u, svh)


def kernel(x, routing_weights, index, u, svh):
    del routing_weights
    return _run(x, index, u, svh)
